# Initial kernel scaffold; baseline (speedup 1.0000x reference)
import jax
import jax.numpy as jnp
from jax.experimental import pallas as pl


def kernel(original_obs, random_noise, W1, b1, W2, b2, W3, b3):
    raise NotImplementedError("write your pallas kernel here")


# speedup vs baseline: 1.6170x; 1.6170x over previous
import functools

import jax
import jax.numpy as jnp
from jax import lax
from jax.experimental import pallas as pl
from jax.experimental.pallas import tpu as pltpu
from jax.experimental.pallas import tpu_sc as plsc

B = 32768
D = 768
H = 256
OUT = 18
OUTP = 128
E = 8
TILE = 256
NT = 136
NP = NT * TILE

NW = 32
RPW = B // NW
XCH = 64
NXCH = RPW // XCH
YCH = 128
NYCH = RPW // YCH


def _route_body(g_ref, t_ref, p_ref, bexp_ref):
    g = g_ref[...]
    e = jnp.zeros(g.shape, jnp.int32)
    for k in range(E - 1):
        e += (g > t_ref[k]).astype(jnp.int32)

    ci = lax.broadcasted_iota(jnp.int32, (128, 128), 0)
    cj = lax.broadcasted_iota(jnp.int32, (128, 128), 1)
    tinc = (ci <= cj).astype(jnp.float32)
    ri = lax.broadcasted_iota(jnp.int32, (256, 256), 0)
    rj = lax.broadcasted_iota(jnp.int32, (256, 256), 1)
    slow = (rj < ri).astype(jnp.float32)

    p_acc = jnp.zeros(g.shape, jnp.float32)
    pstart = jnp.int32(0)
    tstarts = []
    for ex in range(E):
        h = (e == ex).astype(jnp.float32)
        rowcum = jnp.dot(h, tinc, preferred_element_type=jnp.float32)
        excl = rowcum - h
        rowsum = rowcum[:, 127:128]
        rowpre = jnp.dot(slow, jnp.broadcast_to(rowsum, (256, 128)),
                         preferred_element_type=jnp.float32)
        cnt = jnp.sum(h).astype(jnp.int32)
        rank = excl + rowpre
        p_acc += h * (rank + pstart.astype(jnp.float32))
        tstarts.append(pstart // TILE)
        pstart = pstart + ((cnt + TILE - 1) // TILE) * TILE
    p_ref[...] = p_acc.astype(jnp.int32)

    tv = lax.broadcasted_iota(jnp.int32, (1, 256), 1)
    acc = jnp.full((1, 256), -1, jnp.int32)
    for ex in range(E):
        acc += (tv >= tstarts[ex]).astype(jnp.int32)
    bexp_ref[...] = acc


def _route(noise2d, thres):
    return pl.pallas_call(
        _route_body,
        in_specs=[
            pl.BlockSpec(memory_space=pltpu.VMEM),
            pl.BlockSpec(memory_space=pltpu.SMEM),
        ],
        out_specs=[
            pl.BlockSpec(memory_space=pltpu.VMEM),
            pl.BlockSpec(memory_space=pltpu.VMEM),
        ],
        out_shape=[
            jax.ShapeDtypeStruct((256, 128), jnp.int32),
            jax.ShapeDtypeStruct((1, 256), jnp.int32),
        ],
    )(noise2d, thres)


@functools.lru_cache(maxsize=None)
def _sc_kernels():
    mesh = plsc.VectorSubcoreMesh(core_axis_name="c", subcore_axis_name="s")

    @functools.partial(
        pl.kernel,
        out_type=jax.ShapeDtypeStruct((NP, D), jnp.float32),
        mesh=mesh,
        scratch_types=[
            pltpu.VMEM((NXCH, XCH), jnp.int32),
            pltpu.VMEM((XCH, D), jnp.float32),
            pltpu.SemaphoreType.DMA,
        ],
    )
    def scatter_x(x_hbm, p_hbm, xs_hbm, idx_v, xb, sem):
        wid = lax.axis_index("s") * 2 + lax.axis_index("c")
        base = wid * RPW
        pltpu.sync_copy(p_hbm.at[wid], idx_v)
        for ch in range(NXCH):
            pltpu.sync_copy(x_hbm.at[pl.ds(base + ch * XCH, XCH)], xb)
            pltpu.async_copy(xb, xs_hbm.at[idx_v.at[ch]], sem).wait()

    @functools.partial(
        pl.kernel,
        out_type=jax.ShapeDtypeStruct((B, OUTP), jnp.float32),
        mesh=mesh,
        scratch_types=[
            pltpu.VMEM((NYCH, YCH), jnp.int32),
            pltpu.VMEM((YCH, OUTP), jnp.float32),
            pltpu.SemaphoreType.DMA,
        ],
    )
    def gather_y(ys_hbm, p_hbm, out_hbm, idx_v, yb, sem):
        wid = lax.axis_index("s") * 2 + lax.axis_index("c")
        base = wid * RPW
        pltpu.sync_copy(p_hbm.at[wid], idx_v)
        for j in range(NYCH):
            pltpu.async_copy(ys_hbm.at[idx_v.at[j]], yb, sem).wait()
            pltpu.sync_copy(yb, out_hbm.at[pl.ds(base + j * YCH, YCH)])

    return scatter_x, gather_y


def _mlp_body(bexp_ref, xs_ref, w1_ref, b1_ref, w2_ref, b2_ref, w3_ref,
              b3_ref, ys_ref):
    x = xs_ref[...]
    h = jnp.tanh(jnp.dot(x, w1_ref[0], preferred_element_type=jnp.float32)
                 + b1_ref[0])
    h = jnp.tanh(jnp.dot(h, w2_ref[0], preferred_element_type=jnp.float32)
                 + b2_ref[0])
    ys_ref[...] = (jnp.dot(h, w3_ref[0], preferred_element_type=jnp.float32)
                   + b3_ref[0])


def _mlp(bexp, xs, w1, b1, w2, b2, w3p, b3p):
    grid_spec = pltpu.PrefetchScalarGridSpec(
        num_scalar_prefetch=1,
        grid=(NT,),
        in_specs=[
            pl.BlockSpec((TILE, D), lambda t, be: (t, 0)),
            pl.BlockSpec((1, D, H), lambda t, be: (be[t], 0, 0)),
            pl.BlockSpec((1, 1, H), lambda t, be: (be[t], 0, 0)),
            pl.BlockSpec((1, H, H), lambda t, be: (be[t], 0, 0)),
            pl.BlockSpec((1, 1, H), lambda t, be: (be[t], 0, 0)),
            pl.BlockSpec((1, H, OUTP), lambda t, be: (be[t], 0, 0)),
            pl.BlockSpec((1, 1, OUTP), lambda t, be: (be[t], 0, 0)),
        ],
        out_specs=pl.BlockSpec((TILE, OUTP), lambda t, be: (t, 0)),
    )
    return pl.pallas_call(
        _mlp_body,
        grid_spec=grid_spec,
        out_shape=jax.ShapeDtypeStruct((NP, OUTP), jnp.float32),
    )(bexp, xs, w1, b1, w2, b2, w3p, b3p)


def kernel(original_obs, random_noise, W1, b1, W2, b2, W3, b3):
    ps = jnp.arange(1, E, dtype=jnp.float32) / E
    thres = jnp.sqrt(2.0) * jax.scipy.special.erfinv(2.0 * ps - 1.0)

    g2d = random_noise.reshape(256, 128)
    p2d, bexp2d = _route(g2d, thres)
    p = p2d.reshape(B)
    bexp = bexp2d[0, :NT]

    scatter_x, gather_y = _sc_kernels()
    xs = scatter_x(original_obs, p.reshape(NW, NXCH, XCH))

    w3p = jnp.pad(W3, ((0, 0), (0, 0), (0, OUTP - OUT)))
    b3p = jnp.pad(b3, ((0, 0), (0, OUTP - OUT)))
    ys = _mlp(bexp, xs, W1, b1.reshape(E, 1, H), W2, b2.reshape(E, 1, H),
              w3p, b3p.reshape(E, 1, OUTP))

    outp = gather_y(ys, p.reshape(NW, NYCH, YCH))
    return outp[:, :OUT]


# speedup vs baseline: 1.6288x; 1.0073x over previous
import functools

import jax
import jax.numpy as jnp
from jax import lax
from jax.experimental import pallas as pl
from jax.experimental.pallas import tpu as pltpu
from jax.experimental.pallas import tpu_sc as plsc

B = 32768
D = 768
H = 256
OUT = 18
OUTP = 128
E = 8
TILE = 256
NT = 136
NP = NT * TILE

NW = 32
RPW = B // NW
XCH = 64
NXCH = RPW // XCH
YCH = 128
NYCH = RPW // YCH


def _route_body(g_ref, t_ref, p_ref, bexp_ref):
    g = g_ref[...]
    e = jnp.zeros(g.shape, jnp.int32)
    for k in range(E - 1):
        e += (g > t_ref[k]).astype(jnp.int32)

    ci = lax.broadcasted_iota(jnp.int32, (128, 128), 0)
    cj = lax.broadcasted_iota(jnp.int32, (128, 128), 1)
    tinc = (ci <= cj).astype(jnp.float32)
    ri = lax.broadcasted_iota(jnp.int32, (256, 256), 0)
    rj = lax.broadcasted_iota(jnp.int32, (256, 256), 1)
    slow = (rj < ri).astype(jnp.float32)

    p_acc = jnp.zeros(g.shape, jnp.float32)
    pstart = jnp.int32(0)
    tstarts = []
    for ex in range(E):
        h = (e == ex).astype(jnp.float32)
        rowcum = jnp.dot(h, tinc, preferred_element_type=jnp.float32)
        excl = rowcum - h
        rowsum = rowcum[:, 127:128]
        rowpre = jnp.dot(slow, jnp.broadcast_to(rowsum, (256, 128)),
                         preferred_element_type=jnp.float32)
        cnt = jnp.sum(h).astype(jnp.int32)
        rank = excl + rowpre
        p_acc += h * (rank + pstart.astype(jnp.float32))
        tstarts.append(pstart // TILE)
        pstart = pstart + ((cnt + TILE - 1) // TILE) * TILE
    p_ref[...] = p_acc.astype(jnp.int32)

    tv = lax.broadcasted_iota(jnp.int32, (1, 256), 1)
    acc = jnp.full((1, 256), -1, jnp.int32)
    for ex in range(E):
        acc += (tv >= tstarts[ex]).astype(jnp.int32)
    bexp_ref[...] = acc


def _route(noise2d, thres):
    return pl.pallas_call(
        _route_body,
        in_specs=[
            pl.BlockSpec(memory_space=pltpu.VMEM),
            pl.BlockSpec(memory_space=pltpu.SMEM),
        ],
        out_specs=[
            pl.BlockSpec(memory_space=pltpu.VMEM),
            pl.BlockSpec(memory_space=pltpu.VMEM),
        ],
        out_shape=[
            jax.ShapeDtypeStruct((256, 128), jnp.int32),
            jax.ShapeDtypeStruct((1, 256), jnp.int32),
        ],
    )(noise2d, thres)


@functools.lru_cache(maxsize=None)
def _sc_kernels():
    mesh = plsc.VectorSubcoreMesh(core_axis_name="c", subcore_axis_name="s")

    @functools.partial(
        pl.kernel,
        out_type=jax.ShapeDtypeStruct((NP, D), jnp.float32),
        mesh=mesh,
        scratch_types=[
            pltpu.VMEM((NXCH, XCH), jnp.int32),
            pltpu.VMEM((XCH, D), jnp.float32),
            pltpu.SemaphoreType.DMA,
        ],
    )
    def scatter_x(x_hbm, p_hbm, xs_hbm, idx_v, xb, sem):
        wid = lax.axis_index("s") * 2 + lax.axis_index("c")
        base = wid * RPW
        pltpu.sync_copy(p_hbm.at[wid], idx_v)
        for ch in range(NXCH):
            pltpu.sync_copy(x_hbm.at[pl.ds(base + ch * XCH, XCH)], xb)
            pltpu.async_copy(xb, xs_hbm.at[idx_v.at[ch]], sem).wait()

    @functools.partial(
        pl.kernel,
        out_type=jax.ShapeDtypeStruct((B, OUTP), jnp.float32),
        mesh=mesh,
        scratch_types=[
            pltpu.VMEM((NYCH, YCH), jnp.int32),
            pltpu.VMEM((YCH, OUTP), jnp.float32),
            pltpu.SemaphoreType.DMA,
        ],
    )
    def gather_y(ys_hbm, p_hbm, out_hbm, idx_v, yb, sem):
        wid = lax.axis_index("s") * 2 + lax.axis_index("c")
        base = wid * RPW
        pltpu.sync_copy(p_hbm.at[wid], idx_v)
        for j in range(NYCH):
            pltpu.async_copy(ys_hbm.at[idx_v.at[j]], yb, sem).wait()
            pltpu.sync_copy(yb, out_hbm.at[pl.ds(base + j * YCH, YCH)])

    return scatter_x, gather_y


def _mlp_body(bexp_ref, xs_ref, w1_ref, b1_ref, w2_ref, b2_ref, w3_ref,
              b3_ref, ys_ref):
    bf = jnp.bfloat16
    x = xs_ref[...].astype(bf)
    h = jnp.tanh(jnp.dot(x, w1_ref[0].astype(bf),
                         preferred_element_type=jnp.float32) + b1_ref[0])
    h = jnp.tanh(jnp.dot(h.astype(bf), w2_ref[0].astype(bf),
                         preferred_element_type=jnp.float32) + b2_ref[0])
    ys_ref[...] = (jnp.dot(h.astype(bf), w3_ref[0].astype(bf),
                           preferred_element_type=jnp.float32) + b3_ref[0])


def _mlp(bexp, xs, w1, b1, w2, b2, w3p, b3p):
    grid_spec = pltpu.PrefetchScalarGridSpec(
        num_scalar_prefetch=1,
        grid=(NT,),
        in_specs=[
            pl.BlockSpec((TILE, D), lambda t, be: (t, 0)),
            pl.BlockSpec((1, D, H), lambda t, be: (be[t], 0, 0)),
            pl.BlockSpec((1, 1, H), lambda t, be: (be[t], 0, 0)),
            pl.BlockSpec((1, H, H), lambda t, be: (be[t], 0, 0)),
            pl.BlockSpec((1, 1, H), lambda t, be: (be[t], 0, 0)),
            pl.BlockSpec((1, H, OUTP), lambda t, be: (be[t], 0, 0)),
            pl.BlockSpec((1, 1, OUTP), lambda t, be: (be[t], 0, 0)),
        ],
        out_specs=pl.BlockSpec((TILE, OUTP), lambda t, be: (t, 0)),
    )
    return pl.pallas_call(
        _mlp_body,
        grid_spec=grid_spec,
        out_shape=jax.ShapeDtypeStruct((NP, OUTP), jnp.float32),
    )(bexp, xs, w1, b1, w2, b2, w3p, b3p)


def kernel(original_obs, random_noise, W1, b1, W2, b2, W3, b3):
    ps = jnp.arange(1, E, dtype=jnp.float32) / E
    thres = jnp.sqrt(2.0) * jax.scipy.special.erfinv(2.0 * ps - 1.0)

    g2d = random_noise.reshape(256, 128)
    p2d, bexp2d = _route(g2d, thres)
    p = p2d.reshape(B)
    bexp = bexp2d[0, :NT]

    scatter_x, gather_y = _sc_kernels()
    xs = scatter_x(original_obs, p.reshape(NW, NXCH, XCH))

    w3p = jnp.pad(W3, ((0, 0), (0, 0), (0, OUTP - OUT)))
    b3p = jnp.pad(b3, ((0, 0), (0, OUTP - OUT)))
    ys = _mlp(bexp, xs, W1, b1.reshape(E, 1, H), W2, b2.reshape(E, 1, H),
              w3p, b3p.reshape(E, 1, OUTP))

    outp = gather_y(ys, p.reshape(NW, NYCH, YCH))
    return outp[:, :OUT]


# speedup vs baseline: 1.9368x; 1.1891x over previous
import functools

import jax
import jax.numpy as jnp
from jax import lax
from jax.experimental import pallas as pl
from jax.experimental.pallas import tpu as pltpu
from jax.experimental.pallas import tpu_sc as plsc

B = 32768
D = 768
H = 256
OUT = 18
OUTP = 128
E = 8
TILE = 512
NT = 72
NP = NT * TILE

NW = 32
RPW = B // NW
XCH = 64
NXCH = RPW // XCH
YCH = 128
NYCH = RPW // YCH


def _route_body(g_ref, t_ref, p_ref, bexp_ref):
    g = g_ref[...]
    e = jnp.zeros(g.shape, jnp.int32)
    for k in range(E - 1):
        e += (g > t_ref[k]).astype(jnp.int32)

    ci = lax.broadcasted_iota(jnp.int32, (128, 128), 0)
    cj = lax.broadcasted_iota(jnp.int32, (128, 128), 1)
    tinc = (ci <= cj).astype(jnp.float32)
    ri = lax.broadcasted_iota(jnp.int32, (256, 256), 0)
    rj = lax.broadcasted_iota(jnp.int32, (256, 256), 1)
    slow = (rj < ri).astype(jnp.float32)

    p_acc = jnp.zeros(g.shape, jnp.float32)
    pstart = jnp.int32(0)
    tstarts = []
    for ex in range(E):
        h = (e == ex).astype(jnp.float32)
        rowcum = jnp.dot(h, tinc, preferred_element_type=jnp.float32)
        excl = rowcum - h
        rowsum = rowcum[:, 127:128]
        rowpre = jnp.dot(slow, jnp.broadcast_to(rowsum, (256, 128)),
                         preferred_element_type=jnp.float32)
        cnt = jnp.sum(h).astype(jnp.int32)
        rank = excl + rowpre
        p_acc += h * (rank + pstart.astype(jnp.float32))
        tstarts.append(pstart // TILE)
        pstart = pstart + ((cnt + TILE - 1) // TILE) * TILE
    p_ref[...] = p_acc.astype(jnp.int32)

    tv = lax.broadcasted_iota(jnp.int32, (1, 256), 1)
    acc = jnp.full((1, 256), -1, jnp.int32)
    for ex in range(E):
        acc += (tv >= tstarts[ex]).astype(jnp.int32)
    bexp_ref[...] = acc


def _route(noise2d, thres):
    return pl.pallas_call(
        _route_body,
        in_specs=[
            pl.BlockSpec(memory_space=pltpu.VMEM),
            pl.BlockSpec(memory_space=pltpu.SMEM),
        ],
        out_specs=[
            pl.BlockSpec(memory_space=pltpu.VMEM),
            pl.BlockSpec(memory_space=pltpu.VMEM),
        ],
        out_shape=[
            jax.ShapeDtypeStruct((256, 128), jnp.int32),
            jax.ShapeDtypeStruct((1, 256), jnp.int32),
        ],
    )(noise2d, thres)


@functools.lru_cache(maxsize=None)
def _sc_kernels():
    mesh = plsc.VectorSubcoreMesh(core_axis_name="c", subcore_axis_name="s")

    @functools.partial(
        pl.kernel,
        out_type=jax.ShapeDtypeStruct((NP, D), jnp.float32),
        mesh=mesh,
        scratch_types=[
            pltpu.VMEM((NXCH, XCH), jnp.int32),
            pltpu.VMEM((XCH, D), jnp.float32),
            pltpu.SemaphoreType.DMA,
        ],
    )
    def scatter_x(x_hbm, p_hbm, xs_hbm, idx_v, xb, sem):
        wid = lax.axis_index("s") * 2 + lax.axis_index("c")
        base = wid * RPW
        pltpu.sync_copy(p_hbm.at[wid], idx_v)
        for ch in range(NXCH):
            pltpu.sync_copy(x_hbm.at[pl.ds(base + ch * XCH, XCH)], xb)
            pltpu.async_copy(xb, xs_hbm.at[idx_v.at[ch]], sem).wait()

    @functools.partial(
        pl.kernel,
        out_type=jax.ShapeDtypeStruct((B, OUTP), jnp.float32),
        mesh=mesh,
        scratch_types=[
            pltpu.VMEM((NYCH, YCH), jnp.int32),
            pltpu.VMEM((YCH, OUTP), jnp.float32),
            pltpu.SemaphoreType.DMA,
        ],
    )
    def gather_y(ys_hbm, p_hbm, out_hbm, idx_v, yb, sem):
        wid = lax.axis_index("s") * 2 + lax.axis_index("c")
        base = wid * RPW
        pltpu.sync_copy(p_hbm.at[wid], idx_v)
        for j in range(NYCH):
            pltpu.async_copy(ys_hbm.at[idx_v.at[j]], yb, sem).wait()
            pltpu.sync_copy(yb, out_hbm.at[pl.ds(base + j * YCH, YCH)])

    return scatter_x, gather_y


def _mlp_body(bexp_ref, xs_ref, w1_ref, b1_ref, w2_ref, b2_ref, w3_ref,
              b3_ref, ys_ref):
    bf = jnp.bfloat16
    x = xs_ref[...].astype(bf)
    h = jnp.tanh(jnp.dot(x, w1_ref[0],
                         preferred_element_type=jnp.float32) + b1_ref[0])
    h = jnp.tanh(jnp.dot(h.astype(bf), w2_ref[0],
                         preferred_element_type=jnp.float32) + b2_ref[0])
    ys_ref[...] = (jnp.dot(h.astype(bf), w3_ref[0],
                           preferred_element_type=jnp.float32) + b3_ref[0])


def _mlp(bexp, xs, w1, b1, w2, b2, w3p, b3p):
    grid_spec = pltpu.PrefetchScalarGridSpec(
        num_scalar_prefetch=1,
        grid=(NT,),
        in_specs=[
            pl.BlockSpec((TILE, D), lambda t, be: (t, 0)),
            pl.BlockSpec((1, D, H), lambda t, be: (be[t], 0, 0)),
            pl.BlockSpec((1, 1, H), lambda t, be: (be[t], 0, 0)),
            pl.BlockSpec((1, H, H), lambda t, be: (be[t], 0, 0)),
            pl.BlockSpec((1, 1, H), lambda t, be: (be[t], 0, 0)),
            pl.BlockSpec((1, H, OUTP), lambda t, be: (be[t], 0, 0)),
            pl.BlockSpec((1, 1, OUTP), lambda t, be: (be[t], 0, 0)),
        ],
        out_specs=pl.BlockSpec((TILE, OUTP), lambda t, be: (t, 0)),
    )
    return pl.pallas_call(
        _mlp_body,
        grid_spec=grid_spec,
        out_shape=jax.ShapeDtypeStruct((NP, OUTP), jnp.float32),
    )(bexp, xs, w1, b1, w2, b2, w3p, b3p)


def kernel(original_obs, random_noise, W1, b1, W2, b2, W3, b3):
    ps = jnp.arange(1, E, dtype=jnp.float32) / E
    thres = jnp.sqrt(2.0) * jax.scipy.special.erfinv(2.0 * ps - 1.0)

    g2d = random_noise.reshape(256, 128)
    p2d, bexp2d = _route(g2d, thres)
    p = p2d.reshape(B)
    bexp = bexp2d[0, :NT]

    scatter_x, gather_y = _sc_kernels()
    xs = scatter_x(original_obs, p.reshape(NW, NXCH, XCH))

    bf = jnp.bfloat16
    w3p = jnp.pad(W3, ((0, 0), (0, 0), (0, OUTP - OUT))).astype(bf)
    b3p = jnp.pad(b3, ((0, 0), (0, OUTP - OUT)))
    ys = _mlp(bexp, xs, W1.astype(bf), b1.reshape(E, 1, H), W2.astype(bf),
              b2.reshape(E, 1, H), w3p, b3p.reshape(E, 1, OUTP))

    outp = gather_y(ys, p.reshape(NW, NYCH, YCH))
    return outp[:, :OUT]


# speedup vs baseline: 2.0134x; 1.0396x over previous
import functools

import jax
import jax.numpy as jnp
from jax import lax
from jax.experimental import pallas as pl
from jax.experimental.pallas import tpu as pltpu
from jax.experimental.pallas import tpu_sc as plsc

B = 32768
D = 768
H = 256
OUT = 18
OUTP = 128
E = 8
TILE = 512
NT = 72
NP = NT * TILE

NW = 32
RPW = B // NW
XCH = 64
NXCH = RPW // XCH
YCH = 128
NYCH = RPW // YCH


def _route_body(g_ref, t_ref, p_ref, bexp_ref):
    g = g_ref[...]
    e = jnp.zeros(g.shape, jnp.int32)
    for k in range(E - 1):
        e += (g > t_ref[k]).astype(jnp.int32)

    ci = lax.broadcasted_iota(jnp.int32, (128, 128), 0)
    cj = lax.broadcasted_iota(jnp.int32, (128, 128), 1)
    tinc = (ci <= cj).astype(jnp.float32)
    ri = lax.broadcasted_iota(jnp.int32, (256, 256), 0)
    rj = lax.broadcasted_iota(jnp.int32, (256, 256), 1)
    slow = (rj < ri).astype(jnp.float32)

    p_acc = jnp.zeros(g.shape, jnp.float32)
    pstart = jnp.int32(0)
    tstarts = []
    for ex in range(E):
        h = (e == ex).astype(jnp.float32)
        rowcum = jnp.dot(h, tinc, preferred_element_type=jnp.float32)
        excl = rowcum - h
        rowsum = rowcum[:, 127:128]
        rowpre = jnp.dot(slow, jnp.broadcast_to(rowsum, (256, 128)),
                         preferred_element_type=jnp.float32)
        cnt = jnp.sum(h).astype(jnp.int32)
        rank = excl + rowpre
        p_acc += h * (rank + pstart.astype(jnp.float32))
        tstarts.append(pstart // TILE)
        pstart = pstart + ((cnt + TILE - 1) // TILE) * TILE
    p_ref[...] = p_acc.astype(jnp.int32)

    tv = lax.broadcasted_iota(jnp.int32, (1, 256), 1)
    acc = jnp.full((1, 256), -1, jnp.int32)
    for ex in range(E):
        acc += (tv >= tstarts[ex]).astype(jnp.int32)
    bexp_ref[...] = acc


def _route(noise2d, thres):
    return pl.pallas_call(
        _route_body,
        in_specs=[
            pl.BlockSpec(memory_space=pltpu.VMEM),
            pl.BlockSpec(memory_space=pltpu.SMEM),
        ],
        out_specs=[
            pl.BlockSpec(memory_space=pltpu.VMEM),
            pl.BlockSpec(memory_space=pltpu.VMEM),
        ],
        out_shape=[
            jax.ShapeDtypeStruct((256, 128), jnp.int32),
            jax.ShapeDtypeStruct((1, 256), jnp.int32),
        ],
    )(noise2d, thres)


@functools.lru_cache(maxsize=None)
def _sc_kernels():
    mesh = plsc.VectorSubcoreMesh(core_axis_name="c", subcore_axis_name="s")

    @functools.partial(
        pl.kernel,
        out_type=jax.ShapeDtypeStruct((NP, D), jnp.float32),
        mesh=mesh,
        scratch_types=[
            pltpu.VMEM((NXCH, XCH), jnp.int32),
            pltpu.VMEM((XCH, D), jnp.float32),
            pltpu.VMEM((XCH, D), jnp.float32),
            pltpu.SemaphoreType.DMA,
            pltpu.SemaphoreType.DMA,
            pltpu.SemaphoreType.DMA,
            pltpu.SemaphoreType.DMA,
        ],
    )
    def scatter_x(x_hbm, p_hbm, xs_hbm, idx_v, xb0, xb1, sl0, sl1, ss0, ss1):
        wid = lax.axis_index("s") * 2 + lax.axis_index("c")
        base = wid * RPW
        pltpu.sync_copy(p_hbm.at[wid], idx_v)
        xbs = (xb0, xb1)
        sls = (sl0, sl1)
        sss = (ss0, ss1)
        loads = [None] * NXCH
        scats = [None] * NXCH
        loads[0] = pltpu.async_copy(x_hbm.at[pl.ds(base, XCH)], xb0, sl0)
        for ch in range(NXCH):
            loads[ch].wait()
            scats[ch] = pltpu.async_copy(xbs[ch % 2],
                                         xs_hbm.at[idx_v.at[ch]],
                                         sss[ch % 2])
            if ch + 1 < NXCH:
                if ch >= 1:
                    scats[ch - 1].wait()
                loads[ch + 1] = pltpu.async_copy(
                    x_hbm.at[pl.ds(base + (ch + 1) * XCH, XCH)],
                    xbs[(ch + 1) % 2], sls[(ch + 1) % 2])
        scats[NXCH - 2].wait()
        scats[NXCH - 1].wait()

    @functools.partial(
        pl.kernel,
        out_type=jax.ShapeDtypeStruct((B, OUTP), jnp.float32),
        mesh=mesh,
        scratch_types=[
            pltpu.VMEM((NYCH, YCH), jnp.int32),
            pltpu.VMEM((YCH, OUTP), jnp.float32),
            pltpu.SemaphoreType.DMA,
        ],
    )
    def gather_y(ys_hbm, p_hbm, out_hbm, idx_v, yb, sem):
        wid = lax.axis_index("s") * 2 + lax.axis_index("c")
        base = wid * RPW
        pltpu.sync_copy(p_hbm.at[wid], idx_v)
        for j in range(NYCH):
            pltpu.async_copy(ys_hbm.at[idx_v.at[j]], yb, sem).wait()
            pltpu.sync_copy(yb, out_hbm.at[pl.ds(base + j * YCH, YCH)])

    return scatter_x, gather_y


def _mlp_body(bexp_ref, xs_ref, w1_ref, b1_ref, w2_ref, b2_ref, w3_ref,
              b3_ref, ys_ref):
    bf = jnp.bfloat16
    x = xs_ref[...].astype(bf)
    h = jnp.tanh(jnp.dot(x, w1_ref[0],
                         preferred_element_type=jnp.float32) + b1_ref[0])
    h = jnp.tanh(jnp.dot(h.astype(bf), w2_ref[0],
                         preferred_element_type=jnp.float32) + b2_ref[0])
    ys_ref[...] = (jnp.dot(h.astype(bf), w3_ref[0],
                           preferred_element_type=jnp.float32) + b3_ref[0])


def _mlp(bexp, xs, w1, b1, w2, b2, w3p, b3p):
    grid_spec = pltpu.PrefetchScalarGridSpec(
        num_scalar_prefetch=1,
        grid=(NT,),
        in_specs=[
            pl.BlockSpec((TILE, D), lambda t, be: (t, 0)),
            pl.BlockSpec((1, D, H), lambda t, be: (be[t], 0, 0)),
            pl.BlockSpec((1, 1, H), lambda t, be: (be[t], 0, 0)),
            pl.BlockSpec((1, H, H), lambda t, be: (be[t], 0, 0)),
            pl.BlockSpec((1, 1, H), lambda t, be: (be[t], 0, 0)),
            pl.BlockSpec((1, H, OUTP), lambda t, be: (be[t], 0, 0)),
            pl.BlockSpec((1, 1, OUTP), lambda t, be: (be[t], 0, 0)),
        ],
        out_specs=pl.BlockSpec((TILE, OUTP), lambda t, be: (t, 0)),
    )
    return pl.pallas_call(
        _mlp_body,
        grid_spec=grid_spec,
        out_shape=jax.ShapeDtypeStruct((NP, OUTP), jnp.float32),
    )(bexp, xs, w1, b1, w2, b2, w3p, b3p)


def kernel(original_obs, random_noise, W1, b1, W2, b2, W3, b3):
    ps = jnp.arange(1, E, dtype=jnp.float32) / E
    thres = jnp.sqrt(2.0) * jax.scipy.special.erfinv(2.0 * ps - 1.0)

    g2d = random_noise.reshape(256, 128)
    p2d, bexp2d = _route(g2d, thres)
    p = p2d.reshape(B)
    bexp = bexp2d[0, :NT]

    scatter_x, gather_y = _sc_kernels()
    xs = scatter_x(original_obs, p.reshape(NW, NXCH, XCH))

    bf = jnp.bfloat16
    w3p = jnp.pad(W3, ((0, 0), (0, 0), (0, OUTP - OUT))).astype(bf)
    b3p = jnp.pad(b3, ((0, 0), (0, OUTP - OUT)))
    ys = _mlp(bexp, xs, W1.astype(bf), b1.reshape(E, 1, H), W2.astype(bf),
              b2.reshape(E, 1, H), w3p, b3p.reshape(E, 1, OUTP))

    outp = gather_y(ys, p.reshape(NW, NYCH, YCH))
    return outp[:, :OUT]


# speedup vs baseline: 2.1729x; 1.0792x over previous
import functools

import jax
import jax.numpy as jnp
from jax import lax
from jax.experimental import pallas as pl
from jax.experimental.pallas import tpu as pltpu
from jax.experimental.pallas import tpu_sc as plsc

B = 32768
D = 768
H = 256
OUT = 18
OUTP = 128
E = 8
TILE = 1024
NT = 40
NP = NT * TILE

NW = 32
RPW = B // NW
XCH = 64
NXCH = RPW // XCH
YCH = 128
NYCH = RPW // YCH


def _route_body(g_ref, t_ref, p_ref, bexp_ref):
    g = g_ref[...]
    e = jnp.zeros(g.shape, jnp.int32)
    for k in range(E - 1):
        e += (g > t_ref[k]).astype(jnp.int32)

    ci = lax.broadcasted_iota(jnp.int32, (128, 128), 0)
    cj = lax.broadcasted_iota(jnp.int32, (128, 128), 1)
    tinc = (ci <= cj).astype(jnp.float32)
    ri = lax.broadcasted_iota(jnp.int32, (256, 256), 0)
    rj = lax.broadcasted_iota(jnp.int32, (256, 256), 1)
    slow = (rj < ri).astype(jnp.float32)

    p_acc = jnp.zeros(g.shape, jnp.float32)
    pstart = jnp.int32(0)
    tstarts = []
    for ex in range(E):
        h = (e == ex).astype(jnp.float32)
        rowcum = jnp.dot(h, tinc, preferred_element_type=jnp.float32)
        excl = rowcum - h
        rowsum = rowcum[:, 127:128]
        rowpre = jnp.dot(slow, jnp.broadcast_to(rowsum, (256, 128)),
                         preferred_element_type=jnp.float32)
        cnt = jnp.sum(h).astype(jnp.int32)
        rank = excl + rowpre
        p_acc += h * (rank + pstart.astype(jnp.float32))
        tstarts.append(pstart // TILE)
        pstart = pstart + ((cnt + TILE - 1) // TILE) * TILE
    p_ref[...] = p_acc.astype(jnp.int32)

    tv = lax.broadcasted_iota(jnp.int32, (1, 256), 1)
    acc = jnp.full((1, 256), -1, jnp.int32)
    for ex in range(E):
        acc += (tv >= tstarts[ex]).astype(jnp.int32)
    bexp_ref[...] = acc


def _route(noise2d, thres):
    return pl.pallas_call(
        _route_body,
        in_specs=[
            pl.BlockSpec(memory_space=pltpu.VMEM),
            pl.BlockSpec(memory_space=pltpu.SMEM),
        ],
        out_specs=[
            pl.BlockSpec(memory_space=pltpu.VMEM),
            pl.BlockSpec(memory_space=pltpu.VMEM),
        ],
        out_shape=[
            jax.ShapeDtypeStruct((256, 128), jnp.int32),
            jax.ShapeDtypeStruct((1, 256), jnp.int32),
        ],
    )(noise2d, thres)


@functools.lru_cache(maxsize=None)
def _sc_kernels():
    mesh = plsc.VectorSubcoreMesh(core_axis_name="c", subcore_axis_name="s")

    @functools.partial(
        pl.kernel,
        out_type=jax.ShapeDtypeStruct((NP, D), jnp.float32),
        mesh=mesh,
        scratch_types=[
            pltpu.VMEM((NXCH, XCH), jnp.int32),
            pltpu.VMEM((XCH, D), jnp.float32),
            pltpu.VMEM((XCH, D), jnp.float32),
            pltpu.SemaphoreType.DMA,
            pltpu.SemaphoreType.DMA,
            pltpu.SemaphoreType.DMA,
            pltpu.SemaphoreType.DMA,
        ],
    )
    def scatter_x(x_hbm, p_hbm, xs_hbm, idx_v, xb0, xb1, sl0, sl1, ss0, ss1):
        wid = lax.axis_index("s") * 2 + lax.axis_index("c")
        base = wid * RPW
        pltpu.sync_copy(p_hbm.at[wid], idx_v)
        xbs = (xb0, xb1)
        sls = (sl0, sl1)
        sss = (ss0, ss1)
        loads = [None] * NXCH
        scats = [None] * NXCH
        loads[0] = pltpu.async_copy(x_hbm.at[pl.ds(base, XCH)], xb0, sl0)
        for ch in range(NXCH):
            loads[ch].wait()
            scats[ch] = pltpu.async_copy(xbs[ch % 2],
                                         xs_hbm.at[idx_v.at[ch]],
                                         sss[ch % 2])
            if ch + 1 < NXCH:
                if ch >= 1:
                    scats[ch - 1].wait()
                loads[ch + 1] = pltpu.async_copy(
                    x_hbm.at[pl.ds(base + (ch + 1) * XCH, XCH)],
                    xbs[(ch + 1) % 2], sls[(ch + 1) % 2])
        scats[NXCH - 2].wait()
        scats[NXCH - 1].wait()

    @functools.partial(
        pl.kernel,
        out_type=jax.ShapeDtypeStruct((B, OUTP), jnp.float32),
        mesh=mesh,
        scratch_types=[
            pltpu.VMEM((NYCH, YCH), jnp.int32),
            pltpu.VMEM((YCH, OUTP), jnp.float32),
            pltpu.SemaphoreType.DMA,
        ],
    )
    def gather_y(ys_hbm, p_hbm, out_hbm, idx_v, yb, sem):
        wid = lax.axis_index("s") * 2 + lax.axis_index("c")
        base = wid * RPW
        pltpu.sync_copy(p_hbm.at[wid], idx_v)
        for j in range(NYCH):
            pltpu.async_copy(ys_hbm.at[idx_v.at[j]], yb, sem).wait()
            pltpu.sync_copy(yb, out_hbm.at[pl.ds(base + j * YCH, YCH)])

    return scatter_x, gather_y


def _mlp_body(bexp_ref, xs_ref, w1_ref, b1_ref, w2_ref, b2_ref, w3_ref,
              b3_ref, ys_ref):
    bf = jnp.bfloat16
    x = xs_ref[...].astype(bf)
    h = jnp.tanh(jnp.dot(x, w1_ref[0],
                         preferred_element_type=jnp.float32) + b1_ref[0])
    h = jnp.tanh(jnp.dot(h.astype(bf), w2_ref[0],
                         preferred_element_type=jnp.float32) + b2_ref[0])
    ys_ref[...] = (jnp.dot(h.astype(bf), w3_ref[0],
                           preferred_element_type=jnp.float32) + b3_ref[0])


def _mlp(bexp, xs, w1, b1, w2, b2, w3p, b3p):
    grid_spec = pltpu.PrefetchScalarGridSpec(
        num_scalar_prefetch=1,
        grid=(NT,),
        in_specs=[
            pl.BlockSpec((TILE, D), lambda t, be: (t, 0)),
            pl.BlockSpec((1, D, H), lambda t, be: (be[t], 0, 0)),
            pl.BlockSpec((1, 1, H), lambda t, be: (be[t], 0, 0)),
            pl.BlockSpec((1, H, H), lambda t, be: (be[t], 0, 0)),
            pl.BlockSpec((1, 1, H), lambda t, be: (be[t], 0, 0)),
            pl.BlockSpec((1, H, OUTP), lambda t, be: (be[t], 0, 0)),
            pl.BlockSpec((1, 1, OUTP), lambda t, be: (be[t], 0, 0)),
        ],
        out_specs=pl.BlockSpec((TILE, OUTP), lambda t, be: (t, 0)),
    )
    return pl.pallas_call(
        _mlp_body,
        grid_spec=grid_spec,
        out_shape=jax.ShapeDtypeStruct((NP, OUTP), jnp.float32),
    )(bexp, xs, w1, b1, w2, b2, w3p, b3p)


def kernel(original_obs, random_noise, W1, b1, W2, b2, W3, b3):
    ps = jnp.arange(1, E, dtype=jnp.float32) / E
    thres = jnp.sqrt(2.0) * jax.scipy.special.erfinv(2.0 * ps - 1.0)

    g2d = random_noise.reshape(256, 128)
    p2d, bexp2d = _route(g2d, thres)
    p = p2d.reshape(B)
    bexp = bexp2d[0, :NT]

    scatter_x, gather_y = _sc_kernels()
    xs = scatter_x(original_obs, p.reshape(NW, NXCH, XCH))

    bf = jnp.bfloat16
    w3p = jnp.pad(W3, ((0, 0), (0, 0), (0, OUTP - OUT))).astype(bf)
    b3p = jnp.pad(b3, ((0, 0), (0, OUTP - OUT)))
    ys = _mlp(bexp, xs, W1.astype(bf), b1.reshape(E, 1, H), W2.astype(bf),
              b2.reshape(E, 1, H), w3p, b3p.reshape(E, 1, OUTP))

    outp = gather_y(ys, p.reshape(NW, NYCH, YCH))
    return outp[:, :OUT]


# speedup vs baseline: 2.1876x; 1.0067x over previous
import functools

import jax
import jax.numpy as jnp
from jax import lax
from jax.experimental import pallas as pl
from jax.experimental.pallas import tpu as pltpu
from jax.experimental.pallas import tpu_sc as plsc

B = 32768
D = 768
H = 256
OUT = 18
OUTP = 128
E = 8
TILE = 2048
NT = 24
NP = NT * TILE

NW = 32
RPW = B // NW
XCH = 64
NXCH = RPW // XCH
YCH = 128
NYCH = RPW // YCH


def _route_body(g_ref, t_ref, p_ref, bexp_ref):
    g = g_ref[...]
    e = jnp.zeros(g.shape, jnp.int32)
    for k in range(E - 1):
        e += (g > t_ref[k]).astype(jnp.int32)

    ci = lax.broadcasted_iota(jnp.int32, (128, 128), 0)
    cj = lax.broadcasted_iota(jnp.int32, (128, 128), 1)
    tinc = (ci <= cj).astype(jnp.float32)
    ri = lax.broadcasted_iota(jnp.int32, (256, 256), 0)
    rj = lax.broadcasted_iota(jnp.int32, (256, 256), 1)
    slow = (rj < ri).astype(jnp.float32)

    p_acc = jnp.zeros(g.shape, jnp.float32)
    pstart = jnp.int32(0)
    tstarts = []
    for ex in range(E):
        h = (e == ex).astype(jnp.float32)
        rowcum = jnp.dot(h, tinc, preferred_element_type=jnp.float32)
        excl = rowcum - h
        rowsum = rowcum[:, 127:128]
        rowpre = jnp.dot(slow, jnp.broadcast_to(rowsum, (256, 128)),
                         preferred_element_type=jnp.float32)
        cnt = jnp.sum(h).astype(jnp.int32)
        rank = excl + rowpre
        p_acc += h * (rank + pstart.astype(jnp.float32))
        tstarts.append(pstart // TILE)
        pstart = pstart + ((cnt + TILE - 1) // TILE) * TILE
    p_ref[...] = p_acc.astype(jnp.int32)

    tv = lax.broadcasted_iota(jnp.int32, (1, 256), 1)
    acc = jnp.full((1, 256), -1, jnp.int32)
    for ex in range(E):
        acc += (tv >= tstarts[ex]).astype(jnp.int32)
    bexp_ref[...] = acc


def _route(noise2d, thres):
    return pl.pallas_call(
        _route_body,
        in_specs=[
            pl.BlockSpec(memory_space=pltpu.VMEM),
            pl.BlockSpec(memory_space=pltpu.SMEM),
        ],
        out_specs=[
            pl.BlockSpec(memory_space=pltpu.VMEM),
            pl.BlockSpec(memory_space=pltpu.VMEM),
        ],
        out_shape=[
            jax.ShapeDtypeStruct((256, 128), jnp.int32),
            jax.ShapeDtypeStruct((1, 256), jnp.int32),
        ],
    )(noise2d, thres)


@functools.lru_cache(maxsize=None)
def _sc_kernels():
    mesh = plsc.VectorSubcoreMesh(core_axis_name="c", subcore_axis_name="s")

    @functools.partial(
        pl.kernel,
        out_type=jax.ShapeDtypeStruct((NP, D), jnp.float32),
        mesh=mesh,
        scratch_types=[
            pltpu.VMEM((NXCH, XCH), jnp.int32),
            pltpu.VMEM((XCH, D), jnp.float32),
            pltpu.VMEM((XCH, D), jnp.float32),
            pltpu.SemaphoreType.DMA,
            pltpu.SemaphoreType.DMA,
            pltpu.SemaphoreType.DMA,
            pltpu.SemaphoreType.DMA,
        ],
    )
    def scatter_x(x_hbm, p_hbm, xs_hbm, idx_v, xb0, xb1, sl0, sl1, ss0, ss1):
        wid = lax.axis_index("s") * 2 + lax.axis_index("c")
        base = wid * RPW
        pltpu.sync_copy(p_hbm.at[wid], idx_v)
        xbs = (xb0, xb1)
        sls = (sl0, sl1)
        sss = (ss0, ss1)
        loads = [None] * NXCH
        scats = [None] * NXCH
        loads[0] = pltpu.async_copy(x_hbm.at[pl.ds(base, XCH)], xb0, sl0)
        for ch in range(NXCH):
            loads[ch].wait()
            scats[ch] = pltpu.async_copy(xbs[ch % 2],
                                         xs_hbm.at[idx_v.at[ch]],
                                         sss[ch % 2])
            if ch + 1 < NXCH:
                if ch >= 1:
                    scats[ch - 1].wait()
                loads[ch + 1] = pltpu.async_copy(
                    x_hbm.at[pl.ds(base + (ch + 1) * XCH, XCH)],
                    xbs[(ch + 1) % 2], sls[(ch + 1) % 2])
        scats[NXCH - 2].wait()
        scats[NXCH - 1].wait()

    @functools.partial(
        pl.kernel,
        out_type=jax.ShapeDtypeStruct((B, OUTP), jnp.float32),
        mesh=mesh,
        scratch_types=[
            pltpu.VMEM((NYCH, YCH), jnp.int32),
            pltpu.VMEM((YCH, OUTP), jnp.float32),
            pltpu.SemaphoreType.DMA,
        ],
    )
    def gather_y(ys_hbm, p_hbm, out_hbm, idx_v, yb, sem):
        wid = lax.axis_index("s") * 2 + lax.axis_index("c")
        base = wid * RPW
        pltpu.sync_copy(p_hbm.at[wid], idx_v)
        for j in range(NYCH):
            pltpu.async_copy(ys_hbm.at[idx_v.at[j]], yb, sem).wait()
            pltpu.sync_copy(yb, out_hbm.at[pl.ds(base + j * YCH, YCH)])

    return scatter_x, gather_y


def _mlp_body(bexp_ref, xs_ref, w1_ref, b1_ref, w2_ref, b2_ref, w3_ref,
              b3_ref, ys_ref):
    bf = jnp.bfloat16
    x = xs_ref[...].astype(bf)
    h = jnp.tanh(jnp.dot(x, w1_ref[0],
                         preferred_element_type=jnp.float32) + b1_ref[0])
    h = jnp.tanh(jnp.dot(h.astype(bf), w2_ref[0],
                         preferred_element_type=jnp.float32) + b2_ref[0])
    ys_ref[...] = (jnp.dot(h.astype(bf), w3_ref[0],
                           preferred_element_type=jnp.float32) + b3_ref[0])


def _mlp(bexp, xs, w1, b1, w2, b2, w3p, b3p):
    grid_spec = pltpu.PrefetchScalarGridSpec(
        num_scalar_prefetch=1,
        grid=(NT,),
        in_specs=[
            pl.BlockSpec((TILE, D), lambda t, be: (t, 0)),
            pl.BlockSpec((1, D, H), lambda t, be: (be[t], 0, 0)),
            pl.BlockSpec((1, 1, H), lambda t, be: (be[t], 0, 0)),
            pl.BlockSpec((1, H, H), lambda t, be: (be[t], 0, 0)),
            pl.BlockSpec((1, 1, H), lambda t, be: (be[t], 0, 0)),
            pl.BlockSpec((1, H, OUTP), lambda t, be: (be[t], 0, 0)),
            pl.BlockSpec((1, 1, OUTP), lambda t, be: (be[t], 0, 0)),
        ],
        out_specs=pl.BlockSpec((TILE, OUTP), lambda t, be: (t, 0)),
    )
    return pl.pallas_call(
        _mlp_body,
        grid_spec=grid_spec,
        out_shape=jax.ShapeDtypeStruct((NP, OUTP), jnp.float32),
    )(bexp, xs, w1, b1, w2, b2, w3p, b3p)


def kernel(original_obs, random_noise, W1, b1, W2, b2, W3, b3):
    ps = jnp.arange(1, E, dtype=jnp.float32) / E
    thres = jnp.sqrt(2.0) * jax.scipy.special.erfinv(2.0 * ps - 1.0)

    g2d = random_noise.reshape(256, 128)
    p2d, bexp2d = _route(g2d, thres)
    p = p2d.reshape(B)
    bexp = bexp2d[0, :NT]

    scatter_x, gather_y = _sc_kernels()
    xs = scatter_x(original_obs, p.reshape(NW, NXCH, XCH))

    bf = jnp.bfloat16
    w3p = jnp.pad(W3, ((0, 0), (0, 0), (0, OUTP - OUT))).astype(bf)
    b3p = jnp.pad(b3, ((0, 0), (0, OUTP - OUT)))
    ys = _mlp(bexp, xs, W1.astype(bf), b1.reshape(E, 1, H), W2.astype(bf),
              b2.reshape(E, 1, H), w3p, b3p.reshape(E, 1, OUTP))

    outp = gather_y(ys, p.reshape(NW, NYCH, YCH))
    return outp[:, :OUT]
